# Initial kernel scaffold; baseline (speedup 1.0000x reference)
import jax
import jax.numpy as jnp
from jax.experimental import pallas as pl


def kernel(x, W_enc, b_enc, W_dec, b_dec):
    raise NotImplementedError("write your pallas kernel here")


# speedup vs baseline: 26.0959x; 26.0959x over previous
import functools

import jax
import jax.numpy as jnp
from jax.experimental import pallas as pl

K = 32
TILE_M = 256


def _fused_body(x_ref, we_ref, be_ref, wd_ref, bd_ref, recon_ref, sparse_ref):
    x = x_ref[...]
    w_enc = we_ref[...]
    lat = jax.lax.dot_general(
        x, w_enc, (((1,), (1,)), ((), ())),
        preferred_element_type=jnp.float32)
    lat = lat + be_ref[...]

    bits = jax.lax.bitcast_convert_type(lat, jnp.int32)
    key = jnp.where(bits < 0, bits ^ jnp.int32(0x7FFFFFFF), bits)

    cnt0 = jnp.sum((key >= 0).astype(jnp.int32), axis=1, keepdims=True)
    t = jnp.where(cnt0 >= K, jnp.int32(0), jnp.iinfo(jnp.int32).min)
    for b in range(30, -1, -1):
        cand = t + jnp.int32(1 << b)
        cnt = jnp.sum((key >= cand).astype(jnp.int32), axis=1, keepdims=True)
        t = jnp.where(cnt >= K, cand, t)

    sparse = jnp.where(key >= t, lat, 0.0)
    sparse_ref[...] = sparse

    recon = jax.lax.dot_general(
        sparse, wd_ref[...], (((1,), (1,)), ((), ())),
        preferred_element_type=jnp.float32)
    recon_ref[...] = recon + bd_ref[...]


@jax.jit
def kernel(x, W_enc, b_enc, W_dec, b_dec):
    n, d_in = x.shape
    d_lat = W_enc.shape[0]
    grid = (n // TILE_M,)
    recon, sparse = pl.pallas_call(
        _fused_body,
        grid=grid,
        in_specs=[
            pl.BlockSpec((TILE_M, d_in), lambda i: (i, 0)),
            pl.BlockSpec((d_lat, d_in), lambda i: (0, 0)),
            pl.BlockSpec((1, d_lat), lambda i: (0, 0)),
            pl.BlockSpec((d_in, d_lat), lambda i: (0, 0)),
            pl.BlockSpec((1, d_in), lambda i: (0, 0)),
        ],
        out_specs=[
            pl.BlockSpec((TILE_M, d_in), lambda i: (i, 0)),
            pl.BlockSpec((TILE_M, d_lat), lambda i: (i, 0)),
        ],
        out_shape=[
            jax.ShapeDtypeStruct((n, d_in), jnp.float32),
            jax.ShapeDtypeStruct((n, d_lat), jnp.float32),
        ],
    )(x, W_enc, b_enc.reshape(1, -1), W_dec, b_dec.reshape(1, -1))
    return (recon, sparse)


# speedup vs baseline: 29.4590x; 1.1289x over previous
import functools

import jax
import jax.numpy as jnp
from jax.experimental import pallas as pl

K = 32
TILE_M = 256


def _fused_body(x_ref, we_ref, be_ref, wd_ref, bd_ref, recon_ref, sparse_ref):
    x = x_ref[...]
    w_enc = we_ref[...]
    lat = jax.lax.dot_general(
        x, w_enc, (((1,), (1,)), ((), ())),
        preferred_element_type=jnp.float32)
    lat = lat + be_ref[...]

    bits = jax.lax.bitcast_convert_type(lat, jnp.int32)
    key = jnp.where(bits < 0, bits ^ jnp.int32(0x7FFFFFFF), bits)

    cnt0 = jnp.sum((key >= 0).astype(jnp.int32), axis=1, keepdims=True)
    t = jnp.where(cnt0 >= K, jnp.int32(0), jnp.iinfo(jnp.int32).min)
    for b in range(30, 4, -1):
        cand = t + jnp.int32(1 << b)
        cnt = jnp.sum((key >= cand).astype(jnp.int32), axis=1, keepdims=True)
        t = jnp.where(cnt >= K, cand, t)

    sparse = jnp.where(key >= t, lat, 0.0)
    sparse_ref[...] = sparse

    recon = jax.lax.dot_general(
        sparse, wd_ref[...], (((1,), (1,)), ((), ())),
        preferred_element_type=jnp.float32)
    recon_ref[...] = recon + bd_ref[...]


@jax.jit
def kernel(x, W_enc, b_enc, W_dec, b_dec):
    n, d_in = x.shape
    d_lat = W_enc.shape[0]
    grid = (n // TILE_M,)
    recon, sparse = pl.pallas_call(
        _fused_body,
        grid=grid,
        in_specs=[
            pl.BlockSpec((TILE_M, d_in), lambda i: (i, 0)),
            pl.BlockSpec((d_lat, d_in), lambda i: (0, 0)),
            pl.BlockSpec((1, d_lat), lambda i: (0, 0)),
            pl.BlockSpec((d_in, d_lat), lambda i: (0, 0)),
            pl.BlockSpec((1, d_in), lambda i: (0, 0)),
        ],
        out_specs=[
            pl.BlockSpec((TILE_M, d_in), lambda i: (i, 0)),
            pl.BlockSpec((TILE_M, d_lat), lambda i: (i, 0)),
        ],
        out_shape=[
            jax.ShapeDtypeStruct((n, d_in), jnp.float32),
            jax.ShapeDtypeStruct((n, d_lat), jnp.float32),
        ],
    )(x, W_enc, b_enc.reshape(1, -1), W_dec, b_dec.reshape(1, -1))
    return (recon, sparse)


# speedup vs baseline: 33.5312x; 1.1382x over previous
import functools

import jax
import jax.numpy as jnp
from jax.experimental import pallas as pl

K = 32
TILE_M = 256


def _fused_body(x_ref, we_ref, be_ref, wd_ref, bd_ref, recon_ref, sparse_ref):
    x = x_ref[...]
    w_enc = we_ref[...]
    lat = jax.lax.dot_general(
        x, w_enc, (((1,), (1,)), ((), ())),
        preferred_element_type=jnp.float32)
    lat = lat + be_ref[...]

    bits = jax.lax.bitcast_convert_type(lat, jnp.int32)
    key = jnp.where(bits < 0, bits ^ jnp.int32(0x7FFFFFFF), bits)

    xb = lat.astype(jnp.bfloat16)
    m = x.shape[0]
    one_b = jnp.ones((), jnp.bfloat16)
    zero_b = jnp.zeros((), jnp.bfloat16)

    def count_bf16(h):
        q = jnp.where(h < 0, h ^ jnp.int32(0x7FFF), h)
        cb = jax.lax.bitcast_convert_type(q << 16, jnp.float32)
        cb = cb.astype(jnp.bfloat16)
        msk = jnp.where(xb >= cb, one_b, zero_b)
        s = msk[:, :1536] + msk[:, 1536:]
        s = s[:, :768] + s[:, 768:]
        s = s[:, :384] + s[:, 384:]
        s = s[:, :128] + s[:, 128:256] + s[:, 256:]
        return jnp.sum(s.astype(jnp.float32), axis=1, keepdims=True)

    kf = jnp.float32(K)
    zcol = jnp.zeros((m, 1), jnp.int32)
    h = jnp.where(count_bf16(zcol) >= kf, jnp.int32(0), jnp.int32(-32768))
    for b in range(14, -1, -1):
        cand = h + jnp.int32(1 << b)
        h = jnp.where(count_bf16(cand) >= kf, cand, h)

    a0 = jnp.maximum(h, jnp.int32(-32767)) << 16
    lo = a0 - jnp.int32(65536)
    hi = a0 + jnp.int32(262144)
    for _ in range(3):
        mid = lo + ((hi - lo) >> 1)
        cnt = jnp.sum((key >= mid).astype(jnp.int32), axis=1, keepdims=True)
        ge = cnt >= K
        lo = jnp.where(ge, mid, lo)
        hi = jnp.where(ge, hi, mid)

    ri = (jnp.clip(key, lo, lo + jnp.int32(65535)) - lo
          - jnp.int32(32768)).astype(jnp.int16)
    one_s = jnp.ones((), jnp.int16)
    zero_s = jnp.zeros((), jnp.int16)
    base = lo
    for _ in range(13):
        mid = lo + ((hi - lo) >> 1)
        c16 = (mid - base - jnp.int32(32768)).astype(jnp.int16)
        msk = jnp.where(ri >= c16, one_s, zero_s)
        s = msk[:, :1536] + msk[:, 1536:]
        s = s[:, :768] + s[:, 768:]
        s = s[:, :384] + s[:, 384:]
        s = s[:, :128] + s[:, 128:256] + s[:, 256:]
        cnt = jnp.sum(s.astype(jnp.float32), axis=1, keepdims=True)
        ge = cnt >= kf
        lo = jnp.where(ge, mid, lo)
        hi = jnp.where(ge, hi, mid)

    sparse = jnp.where(key >= lo, lat, 0.0)
    sparse_ref[...] = sparse

    recon = jax.lax.dot_general(
        sparse, wd_ref[...], (((1,), (1,)), ((), ())),
        preferred_element_type=jnp.float32)
    recon_ref[...] = recon + bd_ref[...]


@jax.jit
def kernel(x, W_enc, b_enc, W_dec, b_dec):
    n, d_in = x.shape
    d_lat = W_enc.shape[0]
    grid = (n // TILE_M,)
    recon, sparse = pl.pallas_call(
        _fused_body,
        grid=grid,
        in_specs=[
            pl.BlockSpec((TILE_M, d_in), lambda i: (i, 0)),
            pl.BlockSpec((d_lat, d_in), lambda i: (0, 0)),
            pl.BlockSpec((1, d_lat), lambda i: (0, 0)),
            pl.BlockSpec((d_in, d_lat), lambda i: (0, 0)),
            pl.BlockSpec((1, d_in), lambda i: (0, 0)),
        ],
        out_specs=[
            pl.BlockSpec((TILE_M, d_in), lambda i: (i, 0)),
            pl.BlockSpec((TILE_M, d_lat), lambda i: (i, 0)),
        ],
        out_shape=[
            jax.ShapeDtypeStruct((n, d_in), jnp.float32),
            jax.ShapeDtypeStruct((n, d_lat), jnp.float32),
        ],
    )(x, W_enc, b_enc.reshape(1, -1), W_dec, b_dec.reshape(1, -1))
    return (recon, sparse)


# speedup vs baseline: 39.7048x; 1.1841x over previous
import jax
import jax.numpy as jnp
from jax.experimental import pallas as pl

K = 32
TILE_M = 256


def _tree_count(msk):
    s = msk[:, :1536] + msk[:, 1536:]
    s = s[:, :768] + s[:, 768:]
    s = s[:, :384] + s[:, 384:]
    s = s[:, :128] + s[:, 128:256] + s[:, 256:]
    return jnp.sum(s.astype(jnp.float32), axis=1, keepdims=True)


def _fused_body(x_ref, we_ref, be_ref, wd_ref, bd_ref, recon_ref, sparse_ref):
    x = x_ref[...]
    w_enc = we_ref[...]
    lat = jax.lax.dot_general(
        x, w_enc, (((1,), (1,)), ((), ())),
        preferred_element_type=jnp.float32)
    lat = lat + be_ref[...]

    kf = jnp.float32(K)
    xb = lat.astype(jnp.bfloat16)
    one_b = jnp.ones((), jnp.bfloat16)
    zero_b = jnp.zeros((), jnp.bfloat16)

    big = jnp.max(jnp.abs(lat), axis=1, keepdims=True)
    lo = -2.0 * big - 1.0
    hi = 2.0 * big + 1.0
    for _ in range(14):
        mid = 0.5 * (lo + hi)
        cnt = _tree_count(jnp.where(xb >= mid.astype(jnp.bfloat16),
                                    one_b, zero_b))
        ge = cnt >= kf
        lo = jnp.where(ge, mid, lo)
        hi = jnp.where(ge, hi, mid)

    lo1 = lo - (jnp.abs(lo) * jnp.float32(2**-7) + jnp.float32(1e-30))
    hi1 = hi + (jnp.abs(hi) * jnp.float32(2**-7) + jnp.float32(1e-30))

    inv = jnp.float32(1.0 / 65535.0)
    width = jnp.maximum(hi1 - lo1, jnp.float32(1e-30))
    scale = jnp.float32(65535.0) / width
    r = jnp.clip((lat - lo1) * scale, 0.0, 65535.0)
    ri = (r.astype(jnp.int32) - jnp.int32(32768)).astype(jnp.int16)
    one_s = jnp.ones((), jnp.int16)
    zero_s = jnp.zeros((), jnp.int16)

    q_lo = jnp.zeros_like(lat[:, :1], dtype=jnp.int32)
    q_hi = jnp.full_like(q_lo, jnp.int32(65536))
    for _ in range(16):
        q_mid = (q_lo + q_hi) >> 1
        c16 = (q_mid - jnp.int32(32768)).astype(jnp.int16)
        cnt = _tree_count(jnp.where(ri >= c16, one_s, zero_s))
        ge = cnt >= kf
        q_lo = jnp.where(ge, q_mid, q_lo)
        q_hi = jnp.where(ge, q_hi, q_mid)

    v_t = lo1 + q_lo.astype(jnp.float32) * (width * inv)
    sparse = jnp.where(lat >= v_t, lat, 0.0)
    sparse_ref[...] = sparse

    recon = jax.lax.dot_general(
        sparse, wd_ref[...], (((1,), (1,)), ((), ())),
        preferred_element_type=jnp.float32)
    recon_ref[...] = recon + bd_ref[...]


@jax.jit
def kernel(x, W_enc, b_enc, W_dec, b_dec):
    n, d_in = x.shape
    d_lat = W_enc.shape[0]
    grid = (n // TILE_M,)
    recon, sparse = pl.pallas_call(
        _fused_body,
        grid=grid,
        in_specs=[
            pl.BlockSpec((TILE_M, d_in), lambda i: (i, 0)),
            pl.BlockSpec((d_lat, d_in), lambda i: (0, 0)),
            pl.BlockSpec((1, d_lat), lambda i: (0, 0)),
            pl.BlockSpec((d_in, d_lat), lambda i: (0, 0)),
            pl.BlockSpec((1, d_in), lambda i: (0, 0)),
        ],
        out_specs=[
            pl.BlockSpec((TILE_M, d_in), lambda i: (i, 0)),
            pl.BlockSpec((TILE_M, d_lat), lambda i: (i, 0)),
        ],
        out_shape=[
            jax.ShapeDtypeStruct((n, d_in), jnp.float32),
            jax.ShapeDtypeStruct((n, d_lat), jnp.float32),
        ],
    )(x, W_enc, b_enc.reshape(1, -1), W_dec, b_dec.reshape(1, -1))
    return (recon, sparse)


# speedup vs baseline: 40.9272x; 1.0308x over previous
import jax
import jax.numpy as jnp
from jax.experimental import pallas as pl

K = 32
TILE_M = 512


def _count_ge(data, thr, one, zero):
    acc = None
    for c in range(0, 3072, 128):
        part = jnp.where(data[:, c:c + 128] >= thr, one, zero)
        acc = part if acc is None else acc + part
    return jnp.sum(acc.astype(jnp.float32), axis=1, keepdims=True)


def _row_threshold(lat):
    kf = jnp.float32(K)
    xb = lat.astype(jnp.bfloat16)
    one_b = jnp.ones((), jnp.bfloat16)
    zero_b = jnp.zeros((), jnp.bfloat16)

    big = jnp.max(jnp.abs(lat), axis=1, keepdims=True)
    lo = -big
    hi = big * jnp.float32(1.0 + 2**-6) + jnp.float32(1e-30)
    for _ in range(13):
        mid = 0.5 * (lo + hi)
        cnt = _count_ge(xb, mid.astype(jnp.bfloat16), one_b, zero_b)
        ge = cnt >= kf
        lo = jnp.where(ge, mid, lo)
        hi = jnp.where(ge, hi, mid)

    lo1 = lo - (jnp.abs(lo) * jnp.float32(2**-7) + jnp.float32(1e-30))
    hi1 = hi + (jnp.abs(hi) * jnp.float32(2**-7) + jnp.float32(1e-30))

    inv = jnp.float32(1.0 / 65535.0)
    width = jnp.maximum(hi1 - lo1, jnp.float32(1e-30))
    scale = jnp.float32(65535.0) / width
    r = jnp.clip((lat - lo1) * scale, 0.0, 65535.0)
    ri = (r.astype(jnp.int32) - jnp.int32(32768)).astype(jnp.int16)
    one_s = jnp.ones((), jnp.int16)
    zero_s = jnp.zeros((), jnp.int16)

    q_lo = jnp.zeros_like(lat[:, :1], dtype=jnp.int32)
    q_hi = jnp.full_like(q_lo, jnp.int32(65536))
    for _ in range(14):
        q_mid = (q_lo + q_hi) >> 1
        c16 = (q_mid - jnp.int32(32768)).astype(jnp.int16)
        cnt = _count_ge(ri, c16, one_s, zero_s)
        ge = cnt >= kf
        q_lo = jnp.where(ge, q_mid, q_lo)
        q_hi = jnp.where(ge, q_hi, q_mid)

    return lo1 + q_lo.astype(jnp.float32) * (width * inv)


def _fused_body(x_ref, we_ref, be_ref, wd_ref, bd_ref, recon_ref, sparse_ref):
    x = x_ref[...]
    w_enc = we_ref[...]
    lat = jax.lax.dot_general(
        x, w_enc, (((1,), (1,)), ((), ())),
        preferred_element_type=jnp.float32)
    lat = lat + be_ref[...]

    v_t = _row_threshold(lat)
    sparse = jnp.where(lat >= v_t, lat, 0.0)
    sparse_ref[...] = sparse

    recon = jax.lax.dot_general(
        sparse, wd_ref[...], (((1,), (1,)), ((), ())),
        preferred_element_type=jnp.float32)
    recon_ref[...] = recon + bd_ref[...]


@jax.jit
def kernel(x, W_enc, b_enc, W_dec, b_dec):
    n, d_in = x.shape
    d_lat = W_enc.shape[0]
    grid = (n // TILE_M,)
    recon, sparse = pl.pallas_call(
        _fused_body,
        grid=grid,
        in_specs=[
            pl.BlockSpec((TILE_M, d_in), lambda i: (i, 0)),
            pl.BlockSpec((d_lat, d_in), lambda i: (0, 0)),
            pl.BlockSpec((1, d_lat), lambda i: (0, 0)),
            pl.BlockSpec((d_in, d_lat), lambda i: (0, 0)),
            pl.BlockSpec((1, d_in), lambda i: (0, 0)),
        ],
        out_specs=[
            pl.BlockSpec((TILE_M, d_in), lambda i: (i, 0)),
            pl.BlockSpec((TILE_M, d_lat), lambda i: (i, 0)),
        ],
        out_shape=[
            jax.ShapeDtypeStruct((n, d_in), jnp.float32),
            jax.ShapeDtypeStruct((n, d_lat), jnp.float32),
        ],
    )(x, W_enc, b_enc.reshape(1, -1), W_dec, b_dec.reshape(1, -1))
    return (recon, sparse)
